# Initial kernel scaffold; baseline (speedup 1.0000x reference)
import jax
import jax.numpy as jnp
from jax.experimental import pallas as pl


def kernel(x, edge_index, W, b):
    raise NotImplementedError("write your pallas kernel here")


# speedup vs baseline: 6.7463x; 6.7463x over previous
import functools

import jax
import jax.numpy as jnp
from jax import lax
from jax.experimental import pallas as pl
from jax.experimental.pallas import tpu as pltpu
from jax.experimental.pallas import tpu_sc as plsc

N = 10000
N_PAD = 10240
E = 160000
E_PAD = 163840
CH = 256
HALF = 128
NG = 2
CB = CH // NG
NC = 2
NS = 16
ROWS_PER_TILE = N_PAD // NS
P_CHUNKS = 40
P_CHUNK = 256
D_CHUNKS = 10
D_CHUNK = 512
MB = 1024
MB2 = 1000

_mesh = plsc.VectorSubcoreMesh(core_axis_name="c", subcore_axis_name="s")


@functools.partial(
    pl.kernel,
    mesh=_mesh,
    out_type=jax.ShapeDtypeStruct((NC, N_PAD), jnp.float32),
    scratch_types=[
        pltpu.VMEM((D_CHUNK,), jnp.int32),
        pltpu.VMEM((D_CHUNK,), jnp.float32),
        pltpu.VMEM_SHARED((N_PAD,), jnp.float32),
    ],
)
def _sc_degree(dst_hbm, zeros_hbm, ones_hbm, out_hbm, idx_v, ones_v, acc_sh):
    c = lax.axis_index("c")
    s = lax.axis_index("s")
    wid = s * NC + c
    row = pl.ds(s * ROWS_PER_TILE, ROWS_PER_TILE)
    pltpu.sync_copy(ones_hbm, ones_v)
    pltpu.sync_copy(zeros_hbm.at[row], acc_sh.at[row])
    plsc.subcore_barrier()

    @pl.loop(0, D_CHUNKS)
    def _(j):
        pltpu.sync_copy(dst_hbm.at[wid].at[j], idx_v)
        pltpu.sync_copy(ones_v, acc_sh.at[idx_v], add=True)

    plsc.subcore_barrier()
    pltpu.sync_copy(acc_sh.at[row], out_hbm.at[c].at[row])


def _sc_propagate_body(y_hbm, src_hbm, dst_hbm, out_hbm, src_v, dst_v, rows_v,
                       acc_sh):
    c = lax.axis_index("c")
    s = lax.axis_index("s")
    row = pl.ds(s * ROWS_PER_TILE, ROWS_PER_TILE)
    for g in range(NG // NC):
        grp = c * (NG // NC) + g
        pltpu.sync_copy(y_hbm.at[grp].at[row], acc_sh.at[row])
        plsc.subcore_barrier()

        @pl.loop(0, P_CHUNKS)
        def _(j):
            pltpu.sync_copy(src_hbm.at[s].at[j], src_v)
            pltpu.sync_copy(dst_hbm.at[s].at[j], dst_v)
            pltpu.sync_copy(y_hbm.at[grp].at[src_v], rows_v)
            pltpu.sync_copy(rows_v, acc_sh.at[dst_v], add=True)

        plsc.subcore_barrier()
        pltpu.sync_copy(acc_sh.at[row], out_hbm.at[grp].at[row])
        plsc.subcore_barrier()


_sc_propagate = functools.partial(
    pl.kernel,
    mesh=_mesh,
    out_type=jax.ShapeDtypeStruct((NG, N_PAD, CB), jnp.float32),
    scratch_types=[
        pltpu.VMEM((P_CHUNK,), jnp.int32),
        pltpu.VMEM((P_CHUNK,), jnp.int32),
        pltpu.VMEM((P_CHUNK, CB), jnp.float32),
        pltpu.VMEM_SHARED((N_PAD, CB), jnp.float32),
    ],
)(_sc_propagate_body)


def _tc_matmul_body(x_ref, wt_ref, o_ref):
    o_ref[0] = jnp.dot(x_ref[...], wt_ref[0],
                       preferred_element_type=jnp.float32)


def _tc_matmul(x, wt):
    return pl.pallas_call(
        _tc_matmul_body,
        grid=(NG, N_PAD // MB),
        in_specs=[
            pl.BlockSpec((MB, CH), lambda g, i: (i, 0)),
            pl.BlockSpec((1, CH, CB), lambda g, i: (g, 0, 0)),
        ],
        out_specs=pl.BlockSpec((1, MB, CB), lambda g, i: (g, i, 0)),
        out_shape=jax.ShapeDtypeStruct((NG, N_PAD, CB), jnp.float32),
    )(x, wt)


def _tc_deg_finalize_body(degp_ref, dis_ref, dinv_ref):
    deg = degp_ref[0, :] + degp_ref[1, :] + 1.0
    dis_ref[...] = lax.rsqrt(deg)
    dinv_ref[...] = 1.0 / deg


def _tc_deg_finalize(degp):
    return pl.pallas_call(
        _tc_deg_finalize_body,
        out_shape=(
            jax.ShapeDtypeStruct((N_PAD,), jnp.float32),
            jax.ShapeDtypeStruct((N_PAD,), jnp.float32),
        ),
    )(degp)


def _tc_scale_body(y_ref, sc_ref, o_ref):
    o_ref[0] = y_ref[0] * sc_ref[...]


def _tc_scale(y, scale):
    return pl.pallas_call(
        _tc_scale_body,
        grid=(NG, N_PAD // MB),
        in_specs=[
            pl.BlockSpec((1, MB, CB), lambda g, i: (g, i, 0)),
            pl.BlockSpec((MB, 1), lambda g, i: (i, 0)),
        ],
        out_specs=pl.BlockSpec((1, MB, CB), lambda g, i: (g, i, 0)),
        out_shape=jax.ShapeDtypeStruct((NG, N_PAD, CB), jnp.float32),
    )(y, scale)


def _tc_final_body(a_ref, dis_ref, b_ref, o_ref):
    d = dis_ref[...]
    logits = jnp.concatenate([a_ref[g] * d for g in range(NG)], axis=1)
    logits = logits + b_ref[...][None, :]
    m = jnp.max(logits, axis=1, keepdims=True)
    ex = jnp.exp(logits - m)
    lse = jnp.log(jnp.sum(ex, axis=1, keepdims=True)) + m
    o_ref[...] = logits - lse


def _tc_final(a2, dis, b):
    return pl.pallas_call(
        _tc_final_body,
        grid=(N // MB2,),
        in_specs=[
            pl.BlockSpec((NG, MB2, CB), lambda i: (0, i, 0)),
            pl.BlockSpec((MB2, 1), lambda i: (i, 0)),
            pl.BlockSpec((CH,), lambda i: (0,)),
        ],
        out_specs=pl.BlockSpec((MB2, CH), lambda i: (i, 0)),
        out_shape=jax.ShapeDtypeStruct((N, CH), jnp.float32),
    )(a2, dis, b)


def kernel(x, edge_index, W, b):
    src = edge_index[0].astype(jnp.int32)
    dst = edge_index[1].astype(jnp.int32)
    src = jnp.pad(src, (0, E_PAD - E), constant_values=N)
    dst = jnp.pad(dst, (0, E_PAD - E), constant_values=N)
    src_p = src.reshape(NS, P_CHUNKS, P_CHUNK)
    dst_p = dst.reshape(NS, P_CHUNKS, P_CHUNK)
    dst_d = dst.reshape(NC * NS, D_CHUNKS, D_CHUNK)
    zeros1 = jnp.zeros((N_PAD,), jnp.float32)
    ones1 = jnp.ones((D_CHUNK,), jnp.float32)
    wt = W.T.reshape(CH, NG, CB).transpose(1, 0, 2)
    xp = jnp.pad(x, ((0, N_PAD - N), (0, 0)))

    degp = _sc_degree(dst_d, zeros1, ones1)
    z = _tc_matmul(xp, wt)
    dis, dinv = _tc_deg_finalize(degp)
    dis2 = dis.reshape(N_PAD, 1)
    dinv2 = dinv.reshape(N_PAD, 1)
    y0 = _tc_scale(z, dis2)
    a1 = _sc_propagate(y0, src_p, dst_p)
    y1 = _tc_scale(a1, dinv2)
    a2 = _sc_propagate(y1, src_p, dst_p)
    return _tc_final(a2, dis2, b)


# speedup vs baseline: 8.0259x; 1.1897x over previous
import functools

import jax
import jax.numpy as jnp
from jax import lax
from jax.experimental import pallas as pl
from jax.experimental.pallas import tpu as pltpu
from jax.experimental.pallas import tpu_sc as plsc

N = 10000
N_PAD = 10240
E = 160000
E_PAD = 163840
CH = 256
HALF = 128
NG = 2
CB = CH // NG
NC = 2
NS = 16
ROWS_PER_TILE = N_PAD // NS
P_CHUNKS = 80
P_CHUNK = 128
D_CHUNKS = 10
D_CHUNK = 512
MB = 1024
MB2 = 1000

_mesh = plsc.VectorSubcoreMesh(core_axis_name="c", subcore_axis_name="s")


@functools.partial(
    pl.kernel,
    mesh=_mesh,
    out_type=jax.ShapeDtypeStruct((NC, N_PAD), jnp.float32),
    scratch_types=[
        pltpu.VMEM((D_CHUNK,), jnp.int32),
        pltpu.VMEM((D_CHUNK,), jnp.float32),
        pltpu.VMEM_SHARED((N_PAD,), jnp.float32),
    ],
)
def _sc_degree(dst_hbm, zeros_hbm, ones_hbm, out_hbm, idx_v, ones_v, acc_sh):
    c = lax.axis_index("c")
    s = lax.axis_index("s")
    wid = s * NC + c
    row = pl.ds(s * ROWS_PER_TILE, ROWS_PER_TILE)
    pltpu.sync_copy(ones_hbm, ones_v)
    pltpu.sync_copy(zeros_hbm.at[row], acc_sh.at[row])
    plsc.subcore_barrier()

    @pl.loop(0, D_CHUNKS)
    def _(j):
        pltpu.sync_copy(dst_hbm.at[wid].at[j], idx_v)
        pltpu.sync_copy(ones_v, acc_sh.at[idx_v], add=True)

    plsc.subcore_barrier()
    pltpu.sync_copy(acc_sh.at[row], out_hbm.at[c].at[row])


def _sc_propagate_body(y_hbm, src_hbm, dst_hbm, out_hbm,
                       s0, s1, s2, s3, d0, d1, d2, d3, r0, r1, acc_sh,
                       si0, si1, si2, si3, sg0, sg1, ss0, ss1):
    c = lax.axis_index("c")
    s = lax.axis_index("s")
    row = pl.ds(s * ROWS_PER_TILE, ROWS_PER_TILE)
    sv = (s0, s1, s2, s3)
    dv = (d0, d1, d2, d3)
    rv = (r0, r1)
    si = (si0, si1, si2, si3)
    sg = (sg0, sg1)
    ss = (ss0, ss1)

    pltpu.async_copy(src_hbm.at[s].at[0], s0, si0)
    pltpu.async_copy(dst_hbm.at[s].at[0], d0, si0)
    pltpu.async_copy(src_hbm.at[s].at[1], s1, si1)
    pltpu.async_copy(dst_hbm.at[s].at[1], d1, si1)
    pltpu.sync_copy(y_hbm.at[c].at[row], acc_sh.at[row])
    plsc.subcore_barrier()

    @pl.loop(0, P_CHUNKS, step=4)
    def _(j):
        for o in range(4):
            k = j + o
            ib, rb = o % 4, o % 2

            def swait(ib=ib, rb=rb):
                pltpu.make_async_copy(rv[rb], acc_sh.at[dv[ib]], ss[rb]).wait()

            if o < 2:
                pl.when(j > 0)(swait)
            else:
                pltpu.make_async_copy(rv[rb], acc_sh.at[dv[ib - 2]],
                                      ss[rb]).wait()
            @pl.when(k + 2 < P_CHUNKS)
            def _():
                nb = (ib + 2) % 4
                pltpu.async_copy(src_hbm.at[s].at[k + 2], sv[nb], si[nb])
                pltpu.async_copy(dst_hbm.at[s].at[k + 2], dv[nb], si[nb])
            pltpu.make_async_copy(src_hbm.at[s].at[k], sv[ib], si[ib]).wait()
            pltpu.make_async_copy(dst_hbm.at[s].at[k], dv[ib], si[ib]).wait()
            pltpu.async_copy(y_hbm.at[c].at[sv[ib]], rv[rb], sg[rb]).wait()
            pltpu.async_copy(rv[rb], acc_sh.at[dv[ib]], ss[rb], add=True)

    pltpu.make_async_copy(r0, acc_sh.at[d2], ss0).wait()
    pltpu.make_async_copy(r1, acc_sh.at[d3], ss1).wait()
    plsc.subcore_barrier()
    pltpu.sync_copy(acc_sh.at[row], out_hbm.at[c].at[row])


_sc_propagate = functools.partial(
    pl.kernel,
    mesh=_mesh,
    out_type=jax.ShapeDtypeStruct((NG, N_PAD, CB), jnp.float32),
    scratch_types=[
        pltpu.VMEM((P_CHUNK,), jnp.int32),
        pltpu.VMEM((P_CHUNK,), jnp.int32),
        pltpu.VMEM((P_CHUNK,), jnp.int32),
        pltpu.VMEM((P_CHUNK,), jnp.int32),
        pltpu.VMEM((P_CHUNK,), jnp.int32),
        pltpu.VMEM((P_CHUNK,), jnp.int32),
        pltpu.VMEM((P_CHUNK,), jnp.int32),
        pltpu.VMEM((P_CHUNK,), jnp.int32),
        pltpu.VMEM((P_CHUNK, CB), jnp.float32),
        pltpu.VMEM((P_CHUNK, CB), jnp.float32),
        pltpu.VMEM_SHARED((N_PAD, CB), jnp.float32),
        pltpu.SemaphoreType.DMA,
        pltpu.SemaphoreType.DMA,
        pltpu.SemaphoreType.DMA,
        pltpu.SemaphoreType.DMA,
        pltpu.SemaphoreType.DMA,
        pltpu.SemaphoreType.DMA,
        pltpu.SemaphoreType.DMA,
        pltpu.SemaphoreType.DMA,
    ],
)(_sc_propagate_body)


def _tc_matmul_body(x_ref, wt_ref, o_ref):
    o_ref[0] = jnp.dot(x_ref[...], wt_ref[0],
                       preferred_element_type=jnp.float32)


def _tc_matmul(x, wt):
    return pl.pallas_call(
        _tc_matmul_body,
        grid=(NG, N_PAD // MB),
        in_specs=[
            pl.BlockSpec((MB, CH), lambda g, i: (i, 0)),
            pl.BlockSpec((1, CH, CB), lambda g, i: (g, 0, 0)),
        ],
        out_specs=pl.BlockSpec((1, MB, CB), lambda g, i: (g, i, 0)),
        out_shape=jax.ShapeDtypeStruct((NG, N_PAD, CB), jnp.float32),
    )(x, wt)


def _tc_deg_finalize_body(degp_ref, dis_ref, dinv_ref):
    deg = degp_ref[0, :] + degp_ref[1, :] + 1.0
    dis_ref[...] = lax.rsqrt(deg)
    dinv_ref[...] = 1.0 / deg


def _tc_deg_finalize(degp):
    return pl.pallas_call(
        _tc_deg_finalize_body,
        out_shape=(
            jax.ShapeDtypeStruct((N_PAD,), jnp.float32),
            jax.ShapeDtypeStruct((N_PAD,), jnp.float32),
        ),
    )(degp)


def _tc_scale_body(y_ref, sc_ref, o_ref):
    o_ref[0] = y_ref[0] * sc_ref[...]


def _tc_scale(y, scale):
    return pl.pallas_call(
        _tc_scale_body,
        grid=(NG, N_PAD // MB),
        in_specs=[
            pl.BlockSpec((1, MB, CB), lambda g, i: (g, i, 0)),
            pl.BlockSpec((MB, 1), lambda g, i: (i, 0)),
        ],
        out_specs=pl.BlockSpec((1, MB, CB), lambda g, i: (g, i, 0)),
        out_shape=jax.ShapeDtypeStruct((NG, N_PAD, CB), jnp.float32),
    )(y, scale)


def _tc_final_body(a_ref, dis_ref, b_ref, o_ref):
    d = dis_ref[...]
    logits = jnp.concatenate([a_ref[g] * d for g in range(NG)], axis=1)
    logits = logits + b_ref[...][None, :]
    m = jnp.max(logits, axis=1, keepdims=True)
    ex = jnp.exp(logits - m)
    lse = jnp.log(jnp.sum(ex, axis=1, keepdims=True)) + m
    o_ref[...] = logits - lse


def _tc_final(a2, dis, b):
    return pl.pallas_call(
        _tc_final_body,
        grid=(N // MB2,),
        in_specs=[
            pl.BlockSpec((NG, MB2, CB), lambda i: (0, i, 0)),
            pl.BlockSpec((MB2, 1), lambda i: (i, 0)),
            pl.BlockSpec((CH,), lambda i: (0,)),
        ],
        out_specs=pl.BlockSpec((MB2, CH), lambda i: (i, 0)),
        out_shape=jax.ShapeDtypeStruct((N, CH), jnp.float32),
    )(a2, dis, b)


def kernel(x, edge_index, W, b):
    src = edge_index[0].astype(jnp.int32)
    dst = edge_index[1].astype(jnp.int32)
    src = jnp.pad(src, (0, E_PAD - E), constant_values=N)
    dst = jnp.pad(dst, (0, E_PAD - E), constant_values=N)
    src_p = src.reshape(NS, P_CHUNKS, P_CHUNK)
    dst_p = dst.reshape(NS, P_CHUNKS, P_CHUNK)
    dst_d = dst.reshape(NC * NS, D_CHUNKS, D_CHUNK)
    zeros1 = jnp.zeros((N_PAD,), jnp.float32)
    ones1 = jnp.ones((D_CHUNK,), jnp.float32)
    wt = W.T.reshape(CH, NG, CB).transpose(1, 0, 2)
    xp = jnp.pad(x, ((0, N_PAD - N), (0, 0)))

    degp = _sc_degree(dst_d, zeros1, ones1)
    z = _tc_matmul(xp, wt)
    dis, dinv = _tc_deg_finalize(degp)
    dis2 = dis.reshape(N_PAD, 1)
    dinv2 = dinv.reshape(N_PAD, 1)
    y0 = _tc_scale(z, dis2)
    a1 = _sc_propagate(y0, src_p, dst_p)
    y1 = _tc_scale(a1, dinv2)
    a2 = _sc_propagate(y1, src_p, dst_p)
    return _tc_final(a2, dis2, b)


# speedup vs baseline: 8.5430x; 1.0644x over previous
import functools

import jax
import jax.numpy as jnp
from jax import lax
from jax.experimental import pallas as pl
from jax.experimental.pallas import tpu as pltpu
from jax.experimental.pallas import tpu_sc as plsc

N = 10000
N_PAD = 10240
E = 160000
E_PAD = 163840
CH = 256
HALF = 128
NG = 2
CB = CH // NG
NC = 2
NS = 16
ROWS_PER_TILE = N_PAD // NS
P_CHUNKS = 80
P_CHUNK = 128
D_CHUNKS = 10
D_CHUNK = 512
MB = 1024
MB2 = 1000

_mesh = plsc.VectorSubcoreMesh(core_axis_name="c", subcore_axis_name="s")


@functools.partial(
    pl.kernel,
    mesh=_mesh,
    out_type=jax.ShapeDtypeStruct((NC, N_PAD), jnp.float32),
    scratch_types=[
        pltpu.VMEM((D_CHUNK,), jnp.int32),
        pltpu.VMEM((D_CHUNK,), jnp.float32),
        pltpu.VMEM_SHARED((N_PAD,), jnp.float32),
    ],
)
def _sc_degree(dst_hbm, zeros_hbm, ones_hbm, out_hbm, idx_v, ones_v, acc_sh):
    c = lax.axis_index("c")
    s = lax.axis_index("s")
    wid = s * NC + c
    row = pl.ds(s * ROWS_PER_TILE, ROWS_PER_TILE)
    pltpu.sync_copy(ones_hbm, ones_v)
    pltpu.sync_copy(zeros_hbm.at[row], acc_sh.at[row])
    plsc.subcore_barrier()

    @pl.loop(0, D_CHUNKS)
    def _(j):
        pltpu.sync_copy(dst_hbm.at[wid].at[j], idx_v)
        pltpu.sync_copy(ones_v, acc_sh.at[idx_v], add=True)

    plsc.subcore_barrier()
    pltpu.sync_copy(acc_sh.at[row], out_hbm.at[c].at[row])


def _sc_propagate_body(y_hbm, src_hbm, dst_hbm, out_hbm,
                       s0, s1, s2, s3, d0, d1, d2, d3, r0, r1, acc_sh,
                       si0, si1, si2, si3, sg0, sg1, ss0, ss1):
    c = lax.axis_index("c")
    s = lax.axis_index("s")
    row = pl.ds(s * ROWS_PER_TILE, ROWS_PER_TILE)
    sv = (s0, s1, s2, s3)
    dv = (d0, d1, d2, d3)
    rv = (r0, r1)
    si = (si0, si1, si2, si3)
    sg = (sg0, sg1)
    ss = (ss0, ss1)

    pltpu.async_copy(src_hbm.at[s].at[0], s0, si0)
    pltpu.async_copy(dst_hbm.at[s].at[0], d0, si0)
    pltpu.async_copy(src_hbm.at[s].at[1], s1, si1)
    pltpu.async_copy(dst_hbm.at[s].at[1], d1, si1)
    pltpu.sync_copy(y_hbm.at[c].at[row], acc_sh.at[row])
    plsc.subcore_barrier()

    @pl.loop(0, P_CHUNKS, step=4)
    def _(j):
        for o in range(4):
            k = j + o
            ib, rb = o % 4, o % 2
            pb_r = (o + 1) % 2
            pb_i = (o + 3) % 4
            db_i = (o + 2) % 4

            def drain(rb=rb, db_i=db_i):
                pltpu.make_async_copy(rv[rb], acc_sh.at[dv[db_i]],
                                      ss[rb]).wait()

            if o < 2:
                pl.when(j > 0)(drain)
            else:
                drain()
            @pl.when(k + 2 < P_CHUNKS)
            def _():
                pltpu.async_copy(src_hbm.at[s].at[k + 2], sv[db_i], si[db_i])
                pltpu.async_copy(dst_hbm.at[s].at[k + 2], dv[db_i], si[db_i])
            pltpu.make_async_copy(src_hbm.at[s].at[k], sv[ib], si[ib]).wait()
            pltpu.make_async_copy(dst_hbm.at[s].at[k], dv[ib], si[ib]).wait()
            pltpu.async_copy(y_hbm.at[c].at[sv[ib]], rv[rb], sg[rb])

            def finish_prev(pb_r=pb_r, pb_i=pb_i):
                pltpu.make_async_copy(y_hbm.at[c].at[sv[pb_i]], rv[pb_r],
                                      sg[pb_r]).wait()
                pltpu.async_copy(rv[pb_r], acc_sh.at[dv[pb_i]], ss[pb_r],
                                 add=True)

            if o == 0:
                pl.when(j > 0)(finish_prev)
            else:
                finish_prev()

    pltpu.make_async_copy(y_hbm.at[c].at[sv[3]], rv[1], sg[1]).wait()
    pltpu.async_copy(rv[1], acc_sh.at[dv[3]], ss[1], add=True)
    pltpu.make_async_copy(rv[0], acc_sh.at[dv[2]], ss[0]).wait()
    pltpu.make_async_copy(rv[1], acc_sh.at[dv[3]], ss[1]).wait()
    plsc.subcore_barrier()
    pltpu.sync_copy(acc_sh.at[row], out_hbm.at[c].at[row])


_sc_propagate = functools.partial(
    pl.kernel,
    mesh=_mesh,
    out_type=jax.ShapeDtypeStruct((NG, N_PAD, CB), jnp.float32),
    scratch_types=[
        pltpu.VMEM((P_CHUNK,), jnp.int32),
        pltpu.VMEM((P_CHUNK,), jnp.int32),
        pltpu.VMEM((P_CHUNK,), jnp.int32),
        pltpu.VMEM((P_CHUNK,), jnp.int32),
        pltpu.VMEM((P_CHUNK,), jnp.int32),
        pltpu.VMEM((P_CHUNK,), jnp.int32),
        pltpu.VMEM((P_CHUNK,), jnp.int32),
        pltpu.VMEM((P_CHUNK,), jnp.int32),
        pltpu.VMEM((P_CHUNK, CB), jnp.float32),
        pltpu.VMEM((P_CHUNK, CB), jnp.float32),
        pltpu.VMEM_SHARED((N_PAD, CB), jnp.float32),
        pltpu.SemaphoreType.DMA,
        pltpu.SemaphoreType.DMA,
        pltpu.SemaphoreType.DMA,
        pltpu.SemaphoreType.DMA,
        pltpu.SemaphoreType.DMA,
        pltpu.SemaphoreType.DMA,
        pltpu.SemaphoreType.DMA,
        pltpu.SemaphoreType.DMA,
    ],
)(_sc_propagate_body)


def _tc_matmul_body(x_ref, wt_ref, o_ref):
    o_ref[0] = jnp.dot(x_ref[...], wt_ref[0],
                       preferred_element_type=jnp.float32)


def _tc_matmul(x, wt):
    return pl.pallas_call(
        _tc_matmul_body,
        grid=(NG, N_PAD // MB),
        in_specs=[
            pl.BlockSpec((MB, CH), lambda g, i: (i, 0)),
            pl.BlockSpec((1, CH, CB), lambda g, i: (g, 0, 0)),
        ],
        out_specs=pl.BlockSpec((1, MB, CB), lambda g, i: (g, i, 0)),
        out_shape=jax.ShapeDtypeStruct((NG, N_PAD, CB), jnp.float32),
    )(x, wt)


def _tc_deg_finalize_body(degp_ref, dis_ref, dinv_ref):
    deg = degp_ref[0, :] + degp_ref[1, :] + 1.0
    dis_ref[...] = lax.rsqrt(deg)
    dinv_ref[...] = 1.0 / deg


def _tc_deg_finalize(degp):
    return pl.pallas_call(
        _tc_deg_finalize_body,
        out_shape=(
            jax.ShapeDtypeStruct((N_PAD,), jnp.float32),
            jax.ShapeDtypeStruct((N_PAD,), jnp.float32),
        ),
    )(degp)


def _tc_scale_body(y_ref, sc_ref, o_ref):
    o_ref[0] = y_ref[0] * sc_ref[...]


def _tc_scale(y, scale):
    return pl.pallas_call(
        _tc_scale_body,
        grid=(NG, N_PAD // MB),
        in_specs=[
            pl.BlockSpec((1, MB, CB), lambda g, i: (g, i, 0)),
            pl.BlockSpec((MB, 1), lambda g, i: (i, 0)),
        ],
        out_specs=pl.BlockSpec((1, MB, CB), lambda g, i: (g, i, 0)),
        out_shape=jax.ShapeDtypeStruct((NG, N_PAD, CB), jnp.float32),
    )(y, scale)


def _tc_final_body(a_ref, dis_ref, b_ref, o_ref):
    d = dis_ref[...]
    logits = jnp.concatenate([a_ref[g] * d for g in range(NG)], axis=1)
    logits = logits + b_ref[...][None, :]
    m = jnp.max(logits, axis=1, keepdims=True)
    ex = jnp.exp(logits - m)
    lse = jnp.log(jnp.sum(ex, axis=1, keepdims=True)) + m
    o_ref[...] = logits - lse


def _tc_final(a2, dis, b):
    return pl.pallas_call(
        _tc_final_body,
        grid=(N // MB2,),
        in_specs=[
            pl.BlockSpec((NG, MB2, CB), lambda i: (0, i, 0)),
            pl.BlockSpec((MB2, 1), lambda i: (i, 0)),
            pl.BlockSpec((CH,), lambda i: (0,)),
        ],
        out_specs=pl.BlockSpec((MB2, CH), lambda i: (i, 0)),
        out_shape=jax.ShapeDtypeStruct((N, CH), jnp.float32),
    )(a2, dis, b)


def kernel(x, edge_index, W, b):
    src = edge_index[0].astype(jnp.int32)
    dst = edge_index[1].astype(jnp.int32)
    src = jnp.pad(src, (0, E_PAD - E), constant_values=N)
    dst = jnp.pad(dst, (0, E_PAD - E), constant_values=N)
    src_p = src.reshape(NS, P_CHUNKS, P_CHUNK)
    dst_p = dst.reshape(NS, P_CHUNKS, P_CHUNK)
    dst_d = dst.reshape(NC * NS, D_CHUNKS, D_CHUNK)
    zeros1 = jnp.zeros((N_PAD,), jnp.float32)
    ones1 = jnp.ones((D_CHUNK,), jnp.float32)
    wt = W.T.reshape(CH, NG, CB).transpose(1, 0, 2)
    xp = jnp.pad(x, ((0, N_PAD - N), (0, 0)))

    degp = _sc_degree(dst_d, zeros1, ones1)
    z = _tc_matmul(xp, wt)
    dis, dinv = _tc_deg_finalize(degp)
    dis2 = dis.reshape(N_PAD, 1)
    dinv2 = dinv.reshape(N_PAD, 1)
    y0 = _tc_scale(z, dis2)
    a1 = _sc_propagate(y0, src_p, dst_p)
    y1 = _tc_scale(a1, dinv2)
    a2 = _sc_propagate(y1, src_p, dst_p)
    return _tc_final(a2, dis2, b)
